# Initial kernel scaffold; baseline (speedup 1.0000x reference)
import jax
import jax.numpy as jnp
from jax.experimental import pallas as pl


def kernel(inputs, attn_W, attn_b, fc_merged_W, fc_orig_W):
    raise NotImplementedError("write your pallas kernel here")


# speedup vs baseline: 3.9611x; 3.9611x over previous
import functools

import jax
import jax.numpy as jnp
from jax import lax
from jax.experimental import pallas as pl

NEG = -1000000000.0


def _sim_topk_body(x_blk_ref, xt_full_ref, attn_wt_ref, attn_b_ref,
                   idx_ref, attn_ref, *, R, N, K):
    xb = x_blk_ref[0]
    xt = xt_full_ref[0]

    nb = jnp.sqrt(jnp.sum(xb * xb, axis=1, keepdims=True))
    xbn = xb / nb
    nf = jnp.sqrt(jnp.sum(xt * xt, axis=0, keepdims=True))
    xtn = xt / nf

    w = jnp.dot(xbn, xtn, preferred_element_type=jnp.float32)

    iota_n = lax.broadcasted_iota(jnp.int32, (R, N), 1)
    idx_cols = []
    for _ in range(K):
        m = jnp.max(w, axis=1, keepdims=True)
        cand = jnp.where(w == m, iota_n, N)
        sel = jnp.min(cand, axis=1, keepdims=True)
        idx_cols.append(sel)
        w = jnp.where(iota_n == sel, -jnp.inf, w)
    idx_ref[0] = jnp.concatenate(idx_cols, axis=1)

    attn = jnp.dot(xb, attn_wt_ref[...], preferred_element_type=jnp.float32)
    attn_ref[0] = attn + attn_b_ref[...]


def _mask_softmax_body(idx_full_ref, idx_blk_ref, attn_blk_ref,
                       x_full_ref, x_blk_ref, wmt_ref, wot_ref,
                       out_ref, *, R, N, K):
    j = pl.program_id(1)
    r0 = j * R
    idxf = idx_full_ref[0]
    idxb = idx_blk_ref[0]
    attnb = attn_blk_ref[0]

    iota_n = lax.broadcasted_iota(jnp.int32, (R, N), 1)
    row_ids = r0 + lax.broadcasted_iota(jnp.int32, (R, N), 0)

    colmask = jnp.zeros((R, N), dtype=jnp.bool_)
    dense = jnp.full((R, N), NEG, dtype=jnp.float32)
    for k in range(K):
        ik = idxf[:, k].reshape(1, N)
        colmask = colmask | (row_ids == ik)
        dense = jnp.where(iota_n == idxb[:, k:k + 1], attnb[:, k:k + 1], dense)
    dense = jnp.where(colmask, dense, NEG)

    m = jnp.max(dense, axis=1, keepdims=True)
    p = jnp.exp(dense - m)
    adj = p / jnp.sum(p, axis=1, keepdims=True)

    merged = jnp.dot(adj, x_full_ref[0], preferred_element_type=jnp.float32)
    out1 = jnp.maximum(jnp.dot(merged, wmt_ref[...],
                               preferred_element_type=jnp.float32), 0.0)
    out2 = jnp.dot(x_blk_ref[0], wot_ref[...], preferred_element_type=jnp.float32)
    out_ref[0] = out1 + out2


def kernel(inputs, attn_W, attn_b, fc_merged_W, fc_orig_W):
    x = inputs
    B, N, D = x.shape
    K = attn_W.shape[0]
    O = fc_merged_W.shape[0]
    R = 256
    NB = N // R

    xt = jnp.swapaxes(x, 1, 2)
    attn_wt = attn_W.T
    attn_b2 = attn_b.reshape(1, K)
    wmt = fc_merged_W.T
    wot = fc_orig_W.T

    idx, attn = pl.pallas_call(
        functools.partial(_sim_topk_body, R=R, N=N, K=K),
        grid=(B, NB),
        in_specs=[
            pl.BlockSpec((1, R, D), lambda b, j: (b, j, 0)),
            pl.BlockSpec((1, D, N), lambda b, j: (b, 0, 0)),
            pl.BlockSpec((D, K), lambda b, j: (0, 0)),
            pl.BlockSpec((1, K), lambda b, j: (0, 0)),
        ],
        out_specs=[
            pl.BlockSpec((1, R, K), lambda b, j: (b, j, 0)),
            pl.BlockSpec((1, R, K), lambda b, j: (b, j, 0)),
        ],
        out_shape=[
            jax.ShapeDtypeStruct((B, N, K), jnp.int32),
            jax.ShapeDtypeStruct((B, N, K), jnp.float32),
        ],
    )(x, xt, attn_wt, attn_b2)

    out = pl.pallas_call(
        functools.partial(_mask_softmax_body, R=R, N=N, K=K),
        grid=(B, NB),
        in_specs=[
            pl.BlockSpec((1, N, K), lambda b, j: (b, 0, 0)),
            pl.BlockSpec((1, R, K), lambda b, j: (b, j, 0)),
            pl.BlockSpec((1, R, K), lambda b, j: (b, j, 0)),
            pl.BlockSpec((1, N, D), lambda b, j: (b, 0, 0)),
            pl.BlockSpec((1, R, D), lambda b, j: (b, j, 0)),
            pl.BlockSpec((D, O), lambda b, j: (0, 0)),
            pl.BlockSpec((D, O), lambda b, j: (0, 0)),
        ],
        out_specs=pl.BlockSpec((1, R, O), lambda b, j: (b, j, 0)),
        out_shape=jax.ShapeDtypeStruct((B, N, O), jnp.float32),
    )(idx, idx, attn, x, x, wmt, wot)

    return out
